# Initial kernel scaffold; baseline (speedup 1.0000x reference)
import jax
import jax.numpy as jnp
from jax.experimental import pallas as pl


def kernel(x, edge_index, W_enc, b_enc, W_self, W_neigh, b_proc, W_dec, b_dec, W_term, b_term):
    raise NotImplementedError("write your pallas kernel here")


# speedup vs baseline: 19.3410x; 19.3410x over previous
import functools

import jax
import jax.numpy as jnp
from jax import lax
from jax.experimental import pallas as pl
from jax.experimental.pallas import tpu as pltpu
from jax.experimental.pallas import tpu_sc as plsc

NW = 32
NC = 2
LANES = 128


def _sc_body(chunks, x_hbm, src_hbm, dst_hbm, zeros_hbm, ones_hbm,
             s_out, deg_out, src_v, dst_v, vals_v, ones_v, x_sh, s_sh, deg_sh):
    c = lax.axis_index("c")
    s = lax.axis_index("s")
    wid = s * NC + c
    pltpu.sync_copy(src_hbm.at[wid], src_v)
    pltpu.sync_copy(dst_hbm.at[wid], dst_v)
    pltpu.sync_copy(ones_hbm, ones_v)

    @pl.when(s == 0)
    def _():
        pltpu.sync_copy(x_hbm, x_sh)
        pltpu.sync_copy(zeros_hbm, s_sh)
        pltpu.sync_copy(zeros_hbm, deg_sh)

    plsc.subcore_barrier()

    def step(j, carry):
        pltpu.sync_copy(x_sh.at[src_v.at[j]], vals_v)
        pltpu.sync_copy(vals_v, s_sh.at[dst_v.at[j]], add=True)
        pltpu.sync_copy(ones_v, deg_sh.at[dst_v.at[j]], add=True)
        return carry

    lax.fori_loop(0, chunks, step, 0)
    plsc.subcore_barrier()

    @pl.when(s == 0)
    def _():
        pltpu.sync_copy(s_sh, s_out.at[c])
        pltpu.sync_copy(deg_sh, deg_out.at[c])


def _segment_sums(x_flat, src, dst, n):
    e = src.shape[0]
    per_w = e // NW
    chunks = -(-per_w // LANES)
    per_w_pad = chunks * LANES
    nacc = max(n + 1, per_w_pad)
    nacc = -(-nacc // LANES) * LANES
    pad = per_w_pad * NW - e
    src_p = jnp.concatenate([src, jnp.zeros((pad,), jnp.int32)])
    dst_p = jnp.concatenate([dst, jnp.full((pad,), n, jnp.int32)])
    src_r = src_p.reshape(NW, chunks, LANES)
    dst_r = dst_p.reshape(NW, chunks, LANES)
    zeros = jnp.zeros((nacc,), jnp.float32)
    ones = jnp.ones((LANES,), jnp.float32)

    fn = pl.kernel(
        functools.partial(_sc_body, chunks),
        out_type=[jax.ShapeDtypeStruct((NC, nacc), jnp.float32),
                  jax.ShapeDtypeStruct((NC, nacc), jnp.float32)],
        mesh=plsc.VectorSubcoreMesh(core_axis_name="c", subcore_axis_name="s"),
        scratch_types=[
            pltpu.VMEM((chunks, LANES), jnp.int32),
            pltpu.VMEM((chunks, LANES), jnp.int32),
            pltpu.VMEM((LANES,), jnp.float32),
            pltpu.VMEM((LANES,), jnp.float32),
            pltpu.VMEM_SHARED((x_flat.shape[0],), jnp.float32),
            pltpu.VMEM_SHARED((nacc,), jnp.float32),
            pltpu.VMEM_SHARED((nacc,), jnp.float32),
        ],
    )
    return fn(x_flat, src_r, dst_r, zeros, ones)


def _tc_body(n, rows, x_r, s0_r, s1_r, d0_r, d1_r, u_r, be_r, ws_r, wn_r,
             bp_r, wd_r, bd_r, wt_r, bt_r, y_r, h_r, t_r, acc_sum, acc_max):
    i = pl.program_id(0)
    u = u_r[...]
    be = be_r[...]
    ws = ws_r[...]
    wn = wn_r[...]
    a1 = jnp.dot(u, ws, preferred_element_type=jnp.float32)
    a2 = jnp.dot(u, wn, preferred_element_type=jnp.float32)
    a3 = jnp.dot(be, wn, preferred_element_type=jnp.float32)
    a4 = jnp.dot(be, ws, preferred_element_type=jnp.float32) + bp_r[...]

    xb = x_r[...]
    deg = d0_r[...] + d1_r[...]
    cde = jnp.maximum(deg, 1.0)
    r = (s0_r[...] + s1_r[...]) / cde
    m = jnp.minimum(deg, 1.0)
    pre = xb * a1 + r * a2 + m * a3 + a4
    h = jnp.maximum(pre, 0.0)
    h_r[...] = h

    wd = wd_r[...]
    wd1 = wd[:128, :]
    wd2 = wd[128:, :]
    c1 = jnp.dot(u, wd2, preferred_element_type=jnp.float32)
    c0 = jnp.dot(be, wd2, preferred_element_type=jnp.float32)
    logits = (jnp.dot(h, wd1, preferred_element_type=jnp.float32)
              + xb * c1 + c0 + bd_r[...])
    y_r[...] = jax.nn.sigmoid(logits)

    rowid = i * rows + lax.broadcasted_iota(jnp.int32, (rows, 1), 0)
    hm = jnp.where(rowid < n, h, 0.0)
    psum = jnp.sum(hm, axis=0, keepdims=True)
    pmax = jnp.max(hm, axis=0, keepdims=True)

    @pl.when(i == 0)
    def _():
        acc_sum[...] = psum
        acc_max[...] = pmax

    @pl.when(i > 0)
    def _():
        acc_sum[...] = acc_sum[...] + psum
        acc_max[...] = jnp.maximum(acc_max[...], pmax)

    @pl.when(i == pl.num_programs(0) - 1)
    def _():
        wt = wt_r[...]
        hbar = acc_sum[...] * (1.0 / n)
        tt = (jnp.dot(acc_max[...], wt[:128, :], preferred_element_type=jnp.float32)
              + jnp.dot(hbar, wt[128:, :], preferred_element_type=jnp.float32)
              + bt_r[...])
        t_r[...] = jax.nn.sigmoid(tt)


def kernel(x, edge_index, W_enc, b_enc, W_self, W_neigh, b_proc, W_dec,
           b_dec, W_term, b_term):
    n = x.shape[0]
    hdim = W_self.shape[0]
    x_flat = x.reshape(n)

    s_parts, deg_parts = _segment_sums(x_flat, edge_index[0], edge_index[1], n)
    nacc = s_parts.shape[1]

    npad = nacc
    grid = 8
    rows = npad // grid
    x_p = jnp.pad(x, ((0, npad - n), (0, 0)))
    s0 = s_parts[0].reshape(npad, 1)
    s1 = s_parts[1].reshape(npad, 1)
    d0 = deg_parts[0].reshape(npad, 1)
    d1 = deg_parts[1].reshape(npad, 1)
    u = W_enc[0:1, :]
    be = b_enc.reshape(1, hdim)
    bp = b_proc.reshape(1, hdim)
    bd = b_dec.reshape(1, 1)
    bt = b_term.reshape(1, 1)

    col = pl.BlockSpec((rows, 1), lambda i: (i, 0))
    full = lambda a: pl.BlockSpec(a.shape, lambda i: (0,) * a.ndim)
    y_p, h_p, t2 = pl.pallas_call(
        functools.partial(_tc_body, n, rows),
        grid=(grid,),
        in_specs=[col, col, col, col, col,
                  full(u), full(be), full(W_self), full(W_neigh), full(bp),
                  full(W_dec), full(bd), full(W_term), full(bt)],
        out_specs=[col,
                   pl.BlockSpec((rows, hdim), lambda i: (i, 0)),
                   pl.BlockSpec((1, 1), lambda i: (0, 0))],
        out_shape=[jax.ShapeDtypeStruct((npad, 1), jnp.float32),
                   jax.ShapeDtypeStruct((npad, hdim), jnp.float32),
                   jax.ShapeDtypeStruct((1, 1), jnp.float32)],
        scratch_shapes=[pltpu.VMEM((1, hdim), jnp.float32),
                        pltpu.VMEM((1, hdim), jnp.float32)],
    )(x_p, s0, s1, d0, d1, u, be, W_self, W_neigh, bp, W_dec, bd, W_term, bt)

    return (y_p[:n], h_p[:n], t2.reshape(1))


# speedup vs baseline: 20.4623x; 1.0580x over previous
import functools

import jax
import jax.numpy as jnp
from jax import lax
from jax.experimental import pallas as pl
from jax.experimental.pallas import tpu as pltpu
from jax.experimental.pallas import tpu_sc as plsc

NW = 32
NC = 2
LANES = 128


def _sc_body(chunks, x_hbm, src_hbm, dst_hbm, zeros_hbm, ones_hbm,
             s_out, deg_out, src_v, dst_v, vals_v, ones_v, x_sh, s_sh, deg_sh):
    c = lax.axis_index("c")
    s = lax.axis_index("s")
    wid = s * NC + c
    pltpu.sync_copy(src_hbm.at[wid], src_v)
    pltpu.sync_copy(dst_hbm.at[wid], dst_v)
    pltpu.sync_copy(ones_hbm, ones_v)

    @pl.when(s == 0)
    def _():
        pltpu.sync_copy(x_hbm, x_sh)
        pltpu.sync_copy(zeros_hbm, s_sh)
        pltpu.sync_copy(zeros_hbm, deg_sh)

    plsc.subcore_barrier()

    pltpu.sync_copy(x_sh.at[src_v], vals_v)
    pltpu.sync_copy(vals_v, s_sh.at[dst_v], add=True)
    pltpu.sync_copy(ones_v, deg_sh.at[dst_v], add=True)
    plsc.subcore_barrier()

    @pl.when(s == 0)
    def _():
        pltpu.sync_copy(s_sh, s_out.at[c])
        pltpu.sync_copy(deg_sh, deg_out.at[c])


def _segment_sums(x_flat, src, dst, n):
    e = src.shape[0]
    per_w = e // NW
    chunks = -(-per_w // LANES)
    per_w_pad = chunks * LANES
    nacc = max(n + 1, per_w_pad)
    nacc = -(-nacc // LANES) * LANES
    pad = per_w_pad * NW - e
    src_p = jnp.concatenate([src, jnp.zeros((pad,), jnp.int32)])
    dst_p = jnp.concatenate([dst, jnp.full((pad,), n, jnp.int32)])
    src_r = src_p.reshape(NW, per_w_pad)
    dst_r = dst_p.reshape(NW, per_w_pad)
    zeros = jnp.zeros((nacc,), jnp.float32)
    ones = jnp.ones((per_w_pad,), jnp.float32)

    fn = pl.kernel(
        functools.partial(_sc_body, chunks),
        out_type=[jax.ShapeDtypeStruct((NC, nacc), jnp.float32),
                  jax.ShapeDtypeStruct((NC, nacc), jnp.float32)],
        mesh=plsc.VectorSubcoreMesh(core_axis_name="c", subcore_axis_name="s"),
        scratch_types=[
            pltpu.VMEM((per_w_pad,), jnp.int32),
            pltpu.VMEM((per_w_pad,), jnp.int32),
            pltpu.VMEM((per_w_pad,), jnp.float32),
            pltpu.VMEM((per_w_pad,), jnp.float32),
            pltpu.VMEM_SHARED((x_flat.shape[0],), jnp.float32),
            pltpu.VMEM_SHARED((nacc,), jnp.float32),
            pltpu.VMEM_SHARED((nacc,), jnp.float32),
        ],
    )
    return fn(x_flat, src_r, dst_r, zeros, ones)


def _tc_body(n, rows, x_r, s0_r, s1_r, d0_r, d1_r, u_r, be_r, ws_r, wn_r,
             bp_r, wd_r, bd_r, wt_r, bt_r, y_r, h_r, t_r, acc_sum, acc_max):
    i = pl.program_id(0)
    u = u_r[...]
    be = be_r[...]
    ws = ws_r[...]
    wn = wn_r[...]
    a1 = jnp.dot(u, ws, preferred_element_type=jnp.float32)
    a2 = jnp.dot(u, wn, preferred_element_type=jnp.float32)
    a3 = jnp.dot(be, wn, preferred_element_type=jnp.float32)
    a4 = jnp.dot(be, ws, preferred_element_type=jnp.float32) + bp_r[...]

    xb = x_r[...]
    deg = d0_r[...] + d1_r[...]
    cde = jnp.maximum(deg, 1.0)
    r = (s0_r[...] + s1_r[...]) / cde
    m = jnp.minimum(deg, 1.0)
    pre = xb * a1 + r * a2 + m * a3 + a4
    h = jnp.maximum(pre, 0.0)
    h_r[...] = h

    wd = wd_r[...]
    wd1 = wd[:128, :]
    wd2 = wd[128:, :]
    c1 = jnp.dot(u, wd2, preferred_element_type=jnp.float32)
    c0 = jnp.dot(be, wd2, preferred_element_type=jnp.float32)
    logits = (jnp.dot(h, wd1, preferred_element_type=jnp.float32)
              + xb * c1 + c0 + bd_r[...])
    y_r[...] = jax.nn.sigmoid(logits)

    rowid = i * rows + lax.broadcasted_iota(jnp.int32, (rows, 1), 0)
    hm = jnp.where(rowid < n, h, 0.0)
    psum = jnp.sum(hm, axis=0, keepdims=True)
    pmax = jnp.max(hm, axis=0, keepdims=True)

    @pl.when(i == 0)
    def _():
        acc_sum[...] = psum
        acc_max[...] = pmax

    @pl.when(i > 0)
    def _():
        acc_sum[...] = acc_sum[...] + psum
        acc_max[...] = jnp.maximum(acc_max[...], pmax)

    @pl.when(i == pl.num_programs(0) - 1)
    def _():
        wt = wt_r[...]
        hbar = acc_sum[...] * (1.0 / n)
        tt = (jnp.dot(acc_max[...], wt[:128, :], preferred_element_type=jnp.float32)
              + jnp.dot(hbar, wt[128:, :], preferred_element_type=jnp.float32)
              + bt_r[...])
        t_r[...] = jax.nn.sigmoid(tt)


def kernel(x, edge_index, W_enc, b_enc, W_self, W_neigh, b_proc, W_dec,
           b_dec, W_term, b_term):
    n = x.shape[0]
    hdim = W_self.shape[0]
    x_flat = x.reshape(n)

    s_parts, deg_parts = _segment_sums(x_flat, edge_index[0], edge_index[1], n)
    nacc = s_parts.shape[1]

    npad = nacc
    grid = 8
    rows = npad // grid
    x_p = jnp.pad(x, ((0, npad - n), (0, 0)))
    s0 = s_parts[0].reshape(npad, 1)
    s1 = s_parts[1].reshape(npad, 1)
    d0 = deg_parts[0].reshape(npad, 1)
    d1 = deg_parts[1].reshape(npad, 1)
    u = W_enc[0:1, :]
    be = b_enc.reshape(1, hdim)
    bp = b_proc.reshape(1, hdim)
    bd = b_dec.reshape(1, 1)
    bt = b_term.reshape(1, 1)

    col = pl.BlockSpec((rows, 1), lambda i: (i, 0))
    full = lambda a: pl.BlockSpec(a.shape, lambda i: (0,) * a.ndim)
    y_p, h_p, t2 = pl.pallas_call(
        functools.partial(_tc_body, n, rows),
        grid=(grid,),
        in_specs=[col, col, col, col, col,
                  full(u), full(be), full(W_self), full(W_neigh), full(bp),
                  full(W_dec), full(bd), full(W_term), full(bt)],
        out_specs=[col,
                   pl.BlockSpec((rows, hdim), lambda i: (i, 0)),
                   pl.BlockSpec((1, 1), lambda i: (0, 0))],
        out_shape=[jax.ShapeDtypeStruct((npad, 1), jnp.float32),
                   jax.ShapeDtypeStruct((npad, hdim), jnp.float32),
                   jax.ShapeDtypeStruct((1, 1), jnp.float32)],
        scratch_shapes=[pltpu.VMEM((1, hdim), jnp.float32),
                        pltpu.VMEM((1, hdim), jnp.float32)],
    )(x_p, s0, s1, d0, d1, u, be, W_self, W_neigh, bp, W_dec, bd, W_term, bt)

    return (y_p[:n], h_p[:n], t2.reshape(1))


# speedup vs baseline: 22.2200x; 1.0859x over previous
import functools

import jax
import jax.numpy as jnp
from jax import lax
from jax.experimental import pallas as pl
from jax.experimental.pallas import tpu as pltpu
from jax.experimental.pallas import tpu_sc as plsc

NW = 32
NC = 2
LANES = 128


def _sc_body(per_w, nacc, x_hbm, src_hbm, dst_hbm, zeros_hbm,
             s_out, deg_out, x_v, src_v, dst_v, s_part, deg_part,
             sbuf, dbuf, os_v, od_v, s_all, deg_all):
    c = lax.axis_index("c")
    s = lax.axis_index("s")
    wid = s * NC + c
    pltpu.sync_copy(x_hbm, x_v)
    pltpu.sync_copy(src_hbm.at[wid], src_v)
    pltpu.sync_copy(dst_hbm.at[wid], dst_v)
    pltpu.sync_copy(zeros_hbm, s_part)
    pltpu.sync_copy(zeros_hbm, deg_part)

    ones16 = jnp.ones((16,), jnp.float32)
    unroll = 4
    span = 16 * unroll

    def step(j, carry):
        base = j * span
        for k in range(unroll):
            idx_s = src_v[pl.ds(base + 16 * k, 16)]
            idx_d = dst_v[pl.ds(base + 16 * k, 16)]
            vals = plsc.load_gather(x_v, [idx_s])
            plsc.addupdate_scatter(s_part, [idx_d], vals)
            plsc.addupdate_scatter(deg_part, [idx_d], ones16)
        return carry

    lax.fori_loop(0, per_w // span, step, 0)

    pltpu.sync_copy(s_part, s_all.at[s])
    pltpu.sync_copy(deg_part, deg_all.at[s])
    plsc.subcore_barrier()

    w = nacc // 16
    pltpu.sync_copy(s_all.at[:, pl.ds(s * w, w)], sbuf)
    pltpu.sync_copy(deg_all.at[:, pl.ds(s * w, w)], dbuf)

    def red(v, carry):
        sl = pl.ds(v * 16, 16)
        accs = sbuf[0, sl]
        accd = dbuf[0, sl]
        for k in range(1, 16):
            accs = accs + sbuf[k, sl]
            accd = accd + dbuf[k, sl]
        os_v[sl] = accs
        od_v[sl] = accd
        return carry

    lax.fori_loop(0, w // 16, red, 0)
    pltpu.sync_copy(os_v, s_out.at[c, pl.ds(s * w, w)])
    pltpu.sync_copy(od_v, deg_out.at[c, pl.ds(s * w, w)])


def _segment_sums(x_flat, src, dst, n):
    e = src.shape[0]
    per_w = e // NW
    span = 64
    per_w_pad = -(-per_w // span) * span
    nacc = max(n + 1, per_w_pad)
    nacc = -(-nacc // 256) * 256
    pad = per_w_pad * NW - e
    src_p = jnp.concatenate([src, jnp.zeros((pad,), jnp.int32)])
    dst_p = jnp.concatenate([dst, jnp.full((pad,), n, jnp.int32)])
    src_r = src_p.reshape(NW, per_w_pad)
    dst_r = dst_p.reshape(NW, per_w_pad)
    zeros = jnp.zeros((nacc,), jnp.float32)
    w = nacc // 16

    fn = pl.kernel(
        functools.partial(_sc_body, per_w_pad, nacc),
        out_type=[jax.ShapeDtypeStruct((NC, nacc), jnp.float32),
                  jax.ShapeDtypeStruct((NC, nacc), jnp.float32)],
        mesh=plsc.VectorSubcoreMesh(core_axis_name="c", subcore_axis_name="s"),
        compiler_params=pltpu.CompilerParams(needs_layout_passes=False),
        scratch_types=[
            pltpu.VMEM((x_flat.shape[0],), jnp.float32),
            pltpu.VMEM((per_w_pad,), jnp.int32),
            pltpu.VMEM((per_w_pad,), jnp.int32),
            pltpu.VMEM((nacc,), jnp.float32),
            pltpu.VMEM((nacc,), jnp.float32),
            pltpu.VMEM((16, w), jnp.float32),
            pltpu.VMEM((16, w), jnp.float32),
            pltpu.VMEM((w,), jnp.float32),
            pltpu.VMEM((w,), jnp.float32),
            pltpu.VMEM_SHARED((16, nacc), jnp.float32),
            pltpu.VMEM_SHARED((16, nacc), jnp.float32),
        ],
    )
    return fn(x_flat, src_r, dst_r, zeros)


def _tc_body(n, rows, x_r, s0_r, s1_r, d0_r, d1_r, u_r, be_r, ws_r, wn_r,
             bp_r, wd_r, bd_r, wt_r, bt_r, y_r, h_r, t_r, acc_sum, acc_max):
    i = pl.program_id(0)
    u = u_r[...]
    be = be_r[...]
    ws = ws_r[...]
    wn = wn_r[...]
    a1 = jnp.dot(u, ws, preferred_element_type=jnp.float32)
    a2 = jnp.dot(u, wn, preferred_element_type=jnp.float32)
    a3 = jnp.dot(be, wn, preferred_element_type=jnp.float32)
    a4 = jnp.dot(be, ws, preferred_element_type=jnp.float32) + bp_r[...]

    xb = x_r[...]
    deg = d0_r[...] + d1_r[...]
    cde = jnp.maximum(deg, 1.0)
    r = (s0_r[...] + s1_r[...]) / cde
    m = jnp.minimum(deg, 1.0)
    pre = xb * a1 + r * a2 + m * a3 + a4
    h = jnp.maximum(pre, 0.0)
    h_r[...] = h

    wd = wd_r[...]
    wd1 = wd[:128, :]
    wd2 = wd[128:, :]
    c1 = jnp.dot(u, wd2, preferred_element_type=jnp.float32)
    c0 = jnp.dot(be, wd2, preferred_element_type=jnp.float32)
    logits = (jnp.dot(h, wd1, preferred_element_type=jnp.float32)
              + xb * c1 + c0 + bd_r[...])
    y_r[...] = jax.nn.sigmoid(logits)

    rowid = i * rows + lax.broadcasted_iota(jnp.int32, (rows, 1), 0)
    hm = jnp.where(rowid < n, h, 0.0)
    psum = jnp.sum(hm, axis=0, keepdims=True)
    pmax = jnp.max(hm, axis=0, keepdims=True)

    @pl.when(i == 0)
    def _():
        acc_sum[...] = psum
        acc_max[...] = pmax

    @pl.when(i > 0)
    def _():
        acc_sum[...] = acc_sum[...] + psum
        acc_max[...] = jnp.maximum(acc_max[...], pmax)

    @pl.when(i == pl.num_programs(0) - 1)
    def _():
        wt = wt_r[...]
        hbar = acc_sum[...] * (1.0 / n)
        tt = (jnp.dot(acc_max[...], wt[:128, :], preferred_element_type=jnp.float32)
              + jnp.dot(hbar, wt[128:, :], preferred_element_type=jnp.float32)
              + bt_r[...])
        t_r[...] = jax.nn.sigmoid(tt)


def kernel(x, edge_index, W_enc, b_enc, W_self, W_neigh, b_proc, W_dec,
           b_dec, W_term, b_term):
    n = x.shape[0]
    hdim = W_self.shape[0]
    x_flat = x.reshape(n)

    s_parts, deg_parts = _segment_sums(x_flat, edge_index[0], edge_index[1], n)
    nacc = s_parts.shape[1]

    npad = nacc
    grid = 8
    rows = npad // grid
    x_p = jnp.pad(x, ((0, npad - n), (0, 0)))
    s0 = s_parts[0].reshape(npad, 1)
    s1 = s_parts[1].reshape(npad, 1)
    d0 = deg_parts[0].reshape(npad, 1)
    d1 = deg_parts[1].reshape(npad, 1)
    u = W_enc[0:1, :]
    be = b_enc.reshape(1, hdim)
    bp = b_proc.reshape(1, hdim)
    bd = b_dec.reshape(1, 1)
    bt = b_term.reshape(1, 1)

    col = pl.BlockSpec((rows, 1), lambda i: (i, 0))
    full = lambda a: pl.BlockSpec(a.shape, lambda i: (0,) * a.ndim)
    y_p, h_p, t2 = pl.pallas_call(
        functools.partial(_tc_body, n, rows),
        grid=(grid,),
        in_specs=[col, col, col, col, col,
                  full(u), full(be), full(W_self), full(W_neigh), full(bp),
                  full(W_dec), full(bd), full(W_term), full(bt)],
        out_specs=[col,
                   pl.BlockSpec((rows, hdim), lambda i: (i, 0)),
                   pl.BlockSpec((1, 1), lambda i: (0, 0))],
        out_shape=[jax.ShapeDtypeStruct((npad, 1), jnp.float32),
                   jax.ShapeDtypeStruct((npad, hdim), jnp.float32),
                   jax.ShapeDtypeStruct((1, 1), jnp.float32)],
        scratch_shapes=[pltpu.VMEM((1, hdim), jnp.float32),
                        pltpu.VMEM((1, hdim), jnp.float32)],
    )(x_p, s0, s1, d0, d1, u, be, W_self, W_neigh, bp, W_dec, bd, W_term, bt)

    return (y_p[:n], h_p[:n], t2.reshape(1))


# speedup vs baseline: 32.6566x; 1.4697x over previous
import functools

import jax
import jax.numpy as jnp
from jax import lax
from jax.experimental import pallas as pl
from jax.experimental.pallas import tpu as pltpu
from jax.experimental.pallas import tpu_sc as plsc

NW = 32
NC = 2


def _sc_body(per_w, nacc, x_hbm, src_hbm, dst_hbm, zeros_hbm,
             s_out, deg_out, x_v, src_v, dst_v, s_part, deg_part,
             sbuf, dbuf, os_v, od_v, s_all, deg_all):
    c = lax.axis_index("c")
    s = lax.axis_index("s")
    wid = s * NC + c
    pltpu.sync_copy(x_hbm, x_v)
    pltpu.sync_copy(src_hbm.at[pl.ds(wid * per_w, per_w)], src_v)
    pltpu.sync_copy(dst_hbm.at[pl.ds(wid * per_w, per_w)], dst_v)
    pltpu.sync_copy(zeros_hbm, s_part)
    pltpu.sync_copy(zeros_hbm, deg_part)

    ones16 = jnp.ones((16,), jnp.float32)
    unroll = 4
    span = 16 * unroll

    def one_vreg(base):
        idx_s = src_v[pl.ds(base, 16)]
        idx_d = dst_v[pl.ds(base, 16)]
        vals = plsc.load_gather(x_v, [idx_s])
        plsc.addupdate_scatter(s_part, [idx_d], vals)
        plsc.addupdate_scatter(deg_part, [idx_d], ones16)

    def step(j, carry):
        for k in range(unroll):
            one_vreg(j * span + 16 * k)
        return carry

    nfull = per_w // span
    lax.fori_loop(0, nfull, step, 0)
    for k in range((per_w - nfull * span) // 16):
        one_vreg(nfull * span + 16 * k)

    pltpu.sync_copy(s_part, s_all.at[s])
    pltpu.sync_copy(deg_part, deg_all.at[s])
    plsc.subcore_barrier()

    w = nacc // 16
    pltpu.sync_copy(s_all.at[:, pl.ds(s * w, w)], sbuf)
    pltpu.sync_copy(deg_all.at[:, pl.ds(s * w, w)], dbuf)

    def red(v, carry):
        sl = pl.ds(v * 16, 16)
        accs = sbuf[0, sl]
        accd = dbuf[0, sl]
        for k in range(1, 16):
            accs = accs + sbuf[k, sl]
            accd = accd + dbuf[k, sl]
        os_v[sl] = accs
        od_v[sl] = accd
        return carry

    lax.fori_loop(0, w // 16, red, 0)
    pltpu.sync_copy(os_v, s_out.at[c, pl.ds(s * w, w)])
    pltpu.sync_copy(od_v, deg_out.at[c, pl.ds(s * w, w)])


def _segment_sums(x_flat, src, dst, n):
    e = src.shape[0]
    per_w = e // NW
    nacc = -(-n // 256) * 256
    zeros = jnp.zeros((nacc,), jnp.float32)
    w = nacc // 16

    fn = pl.kernel(
        functools.partial(_sc_body, per_w, nacc),
        out_type=[jax.ShapeDtypeStruct((NC, nacc), jnp.float32),
                  jax.ShapeDtypeStruct((NC, nacc), jnp.float32)],
        mesh=plsc.VectorSubcoreMesh(core_axis_name="c", subcore_axis_name="s"),
        compiler_params=pltpu.CompilerParams(needs_layout_passes=False),
        scratch_types=[
            pltpu.VMEM((x_flat.shape[0],), jnp.float32),
            pltpu.VMEM((per_w,), jnp.int32),
            pltpu.VMEM((per_w,), jnp.int32),
            pltpu.VMEM((nacc,), jnp.float32),
            pltpu.VMEM((nacc,), jnp.float32),
            pltpu.VMEM((16, w), jnp.float32),
            pltpu.VMEM((16, w), jnp.float32),
            pltpu.VMEM((w,), jnp.float32),
            pltpu.VMEM((w,), jnp.float32),
            pltpu.VMEM_SHARED((16, nacc), jnp.float32),
            pltpu.VMEM_SHARED((16, nacc), jnp.float32),
        ],
    )
    return fn(x_flat, src, dst, zeros)


def _tc_body(n, hdim, x_r, sp_r, dp_r, u_r, be_r, ws_r, wn_r,
             bp_r, wd_r, bd_r, wt_r, bt_r, y_r, h_r, t_r):
    f32 = jnp.float32
    dnT = (((0,), (1,)), ((), ()))
    dn0 = (((0,), (0,)), ((), ()))
    u = u_r[...]
    be = be_r[...]
    ws = ws_r[...]
    wn = wn_r[...]
    a1 = lax.dot_general(ws, u, dnT, preferred_element_type=f32)
    a2 = lax.dot_general(wn, u, dnT, preferred_element_type=f32)
    a3 = lax.dot_general(wn, be, dnT, preferred_element_type=f32)
    a4 = (lax.dot_general(ws, be, dnT, preferred_element_type=f32)
          + lax.dot_general(bp_r[...], jnp.ones((1, 1), f32), dnT,
                            preferred_element_type=f32))

    x_row = x_r[...].reshape(1, n)
    s_row = sp_r[0:1, :n] + sp_r[1:2, :n]
    deg = dp_r[0:1, :n] + dp_r[1:2, :n]
    cde = jnp.maximum(deg, 1.0)
    r_row = s_row / cde
    m_row = jnp.minimum(deg, 1.0)

    pre = a1 * x_row + a2 * r_row + a3 * m_row + a4
    hT = jnp.maximum(pre, 0.0)

    eye = jnp.eye(hdim, dtype=f32)
    h = lax.dot_general(hT, eye, dn0, preferred_element_type=f32)
    h_r[...] = h

    wd = wd_r[...]
    wd1 = wd[:hdim, :]
    wd2 = wd[hdim:, :]
    c1 = lax.dot_general(u, wd2, (((1,), (0,)), ((), ())),
                         preferred_element_type=f32)
    c0 = lax.dot_general(be, wd2, (((1,), (0,)), ((), ())),
                         preferred_element_type=f32)
    x_col = lax.dot_general(x_row, jnp.ones((1, 1), f32), dn0,
                            preferred_element_type=f32)
    logits = (jnp.dot(h, wd1, preferred_element_type=f32)
              + x_col * c1 + c0 + bd_r[...])
    y_r[...] = jax.nn.sigmoid(logits)

    psum = jnp.sum(hT, axis=1, keepdims=True)
    pmax = jnp.max(hT, axis=1, keepdims=True)
    wt = wt_r[...]
    tt = (lax.dot_general(pmax, wt[:hdim, :], dn0, preferred_element_type=f32)
          + lax.dot_general(psum * (1.0 / n), wt[hdim:, :], dn0,
                            preferred_element_type=f32)
          + bt_r[...])
    t_r[...] = jax.nn.sigmoid(tt)


def kernel(x, edge_index, W_enc, b_enc, W_self, W_neigh, b_proc, W_dec,
           b_dec, W_term, b_term):
    n = x.shape[0]
    hdim = W_self.shape[0]
    x_flat = x.reshape(n)

    s_parts, deg_parts = _segment_sums(x_flat, edge_index[0], edge_index[1], n)

    u = W_enc[0:1, :]
    be = b_enc.reshape(1, hdim)
    bp = b_proc.reshape(1, hdim)
    bd = b_dec.reshape(1, 1)
    bt = b_term.reshape(1, 1)

    full = lambda a: pl.BlockSpec(a.shape, lambda: (0,) * a.ndim)
    args = (x_flat, s_parts, deg_parts, u, be, W_self, W_neigh, bp,
            W_dec, bd, W_term, bt)
    y, h, t2 = pl.pallas_call(
        functools.partial(_tc_body, n, hdim),
        in_specs=[full(a) for a in args],
        out_specs=[pl.BlockSpec((n, 1), lambda: (0, 0)),
                   pl.BlockSpec((n, hdim), lambda: (0, 0)),
                   pl.BlockSpec((1, 1), lambda: (0, 0))],
        out_shape=[jax.ShapeDtypeStruct((n, 1), jnp.float32),
                   jax.ShapeDtypeStruct((n, hdim), jnp.float32),
                   jax.ShapeDtypeStruct((1, 1), jnp.float32)],
    )(*args)

    return (y, h, t2.reshape(1))


# speedup vs baseline: 39.7976x; 1.2187x over previous
import functools

import jax
import jax.numpy as jnp
from jax import lax
from jax.experimental import pallas as pl
from jax.experimental.pallas import tpu as pltpu
from jax.experimental.pallas import tpu_sc as plsc

NW = 32
NC = 2


def _sc_body(per_w, nacc, x_hbm, edge_hbm, zeros_hbm,
             s_out, deg_out, x_v, src_v, dst_v, s_part, deg_part,
             sbuf, dbuf, os_v, od_v, s_all, deg_all):
    c = lax.axis_index("c")
    s = lax.axis_index("s")
    wid = s * NC + c
    pltpu.sync_copy(x_hbm, x_v)
    pltpu.sync_copy(edge_hbm.at[0, pl.ds(wid * per_w, per_w)], src_v)
    pltpu.sync_copy(edge_hbm.at[1, pl.ds(wid * per_w, per_w)], dst_v)
    pltpu.sync_copy(zeros_hbm, s_part)
    pltpu.sync_copy(zeros_hbm, deg_part)

    ones16 = jnp.ones((16,), jnp.float32)

    def one_vreg(j):
        base = j * 16
        idx_s = src_v[pl.ds(base, 16)]
        idx_d = dst_v[pl.ds(base, 16)]
        vals = plsc.load_gather(x_v, [idx_s])
        plsc.addupdate_scatter(s_part, [idx_d], vals)
        plsc.addupdate_scatter(deg_part, [idx_d], ones16)

    plsc.parallel_loop(0, per_w // 16, unroll=8)(one_vreg)

    pltpu.sync_copy(s_part, s_all.at[s])
    pltpu.sync_copy(deg_part, deg_all.at[s])
    plsc.subcore_barrier()

    w = nacc // 16
    pltpu.sync_copy(s_all.at[:, pl.ds(s * w, w)], sbuf)
    pltpu.sync_copy(deg_all.at[:, pl.ds(s * w, w)], dbuf)

    def red(v, carry):
        sl = pl.ds(v * 16, 16)
        accs = sbuf[0, sl]
        accd = dbuf[0, sl]
        for k in range(1, 16):
            accs = accs + sbuf[k, sl]
            accd = accd + dbuf[k, sl]
        os_v[sl] = accs
        od_v[sl] = accd
        return carry

    lax.fori_loop(0, w // 16, red, 0)
    pltpu.sync_copy(os_v, s_out.at[c, pl.ds(s * w, w)])
    pltpu.sync_copy(od_v, deg_out.at[c, pl.ds(s * w, w)])


def _segment_sums(x_flat, edge_index, n):
    e = edge_index.shape[1]
    per_w = e // NW
    nacc = -(-n // 256) * 256
    zeros = jnp.zeros((nacc,), jnp.float32)
    w = nacc // 16

    fn = pl.kernel(
        functools.partial(_sc_body, per_w, nacc),
        out_type=[jax.ShapeDtypeStruct((NC, nacc), jnp.float32),
                  jax.ShapeDtypeStruct((NC, nacc), jnp.float32)],
        mesh=plsc.VectorSubcoreMesh(core_axis_name="c", subcore_axis_name="s"),
        compiler_params=pltpu.CompilerParams(needs_layout_passes=False,
                                             use_tc_tiling_on_sc=False),
        scratch_types=[
            pltpu.VMEM((x_flat.shape[0],), jnp.float32),
            pltpu.VMEM((per_w,), jnp.int32),
            pltpu.VMEM((per_w,), jnp.int32),
            pltpu.VMEM((nacc,), jnp.float32),
            pltpu.VMEM((nacc,), jnp.float32),
            pltpu.VMEM((16, w), jnp.float32),
            pltpu.VMEM((16, w), jnp.float32),
            pltpu.VMEM((w,), jnp.float32),
            pltpu.VMEM((w,), jnp.float32),
            pltpu.VMEM_SHARED((16, nacc), jnp.float32),
            pltpu.VMEM_SHARED((16, nacc), jnp.float32),
        ],
    )
    return fn(x_flat, edge_index, zeros)


def _tc_body(n, hdim, x_r, sp_r, dp_r, u_r, be_r, ws_r, wn_r,
             bp_r, wd_r, bd_r, wt_r, bt_r, y_r, h_r, t_r):
    f32 = jnp.float32
    dnT = (((0,), (1,)), ((), ()))
    dn0 = (((0,), (0,)), ((), ()))
    u = u_r[...]
    be = be_r[...]
    ws = ws_r[...]
    wn = wn_r[...]
    a1 = lax.dot_general(ws, u, dnT, preferred_element_type=f32)
    a2 = lax.dot_general(wn, u, dnT, preferred_element_type=f32)
    a3 = lax.dot_general(wn, be, dnT, preferred_element_type=f32)
    a4 = (lax.dot_general(ws, be, dnT, preferred_element_type=f32)
          + lax.dot_general(bp_r[...], jnp.ones((1, 1), f32), dnT,
                            preferred_element_type=f32))

    x_row = x_r[...].reshape(1, n)
    s_row = sp_r[0:1, :n] + sp_r[1:2, :n]
    deg = dp_r[0:1, :n] + dp_r[1:2, :n]
    cde = jnp.maximum(deg, 1.0)
    r_row = s_row / cde
    m_row = jnp.minimum(deg, 1.0)

    pre = a1 * x_row + a2 * r_row + a3 * m_row + a4
    hT = jnp.maximum(pre, 0.0)

    eye = jnp.eye(hdim, dtype=f32)
    h = lax.dot_general(hT, eye, dn0, preferred_element_type=f32)
    h_r[...] = h

    wd = wd_r[...]
    wd1 = wd[:hdim, :]
    wd2 = wd[hdim:, :]
    c1 = lax.dot_general(u, wd2, (((1,), (0,)), ((), ())),
                         preferred_element_type=f32)
    c0 = lax.dot_general(be, wd2, (((1,), (0,)), ((), ())),
                         preferred_element_type=f32)
    x_col = lax.dot_general(x_row, jnp.ones((1, 1), f32), dn0,
                            preferred_element_type=f32)
    logits = (jnp.dot(h, wd1, preferred_element_type=f32)
              + x_col * c1 + c0 + bd_r[...])
    y_r[...] = jax.nn.sigmoid(logits)

    psum = jnp.sum(hT, axis=1, keepdims=True)
    pmax = jnp.max(hT, axis=1, keepdims=True)
    wt = wt_r[...]
    tt = (lax.dot_general(pmax, wt[:hdim, :], dn0, preferred_element_type=f32)
          + lax.dot_general(psum * (1.0 / n), wt[hdim:, :], dn0,
                            preferred_element_type=f32)
          + bt_r[...])
    t_r[...] = jax.nn.sigmoid(tt)


def kernel(x, edge_index, W_enc, b_enc, W_self, W_neigh, b_proc, W_dec,
           b_dec, W_term, b_term):
    n = x.shape[0]
    hdim = W_self.shape[0]
    x_flat = x.reshape(n)

    s_parts, deg_parts = _segment_sums(x_flat, edge_index, n)

    u = W_enc[0:1, :]
    be = b_enc.reshape(1, hdim)
    bp = b_proc.reshape(1, hdim)
    bd = b_dec.reshape(1, 1)
    bt = b_term.reshape(1, 1)

    full = lambda a: pl.BlockSpec(a.shape, lambda: (0,) * a.ndim)
    args = (x_flat, s_parts, deg_parts, u, be, W_self, W_neigh, bp,
            W_dec, bd, W_term, bt)
    y, h, t2 = pl.pallas_call(
        functools.partial(_tc_body, n, hdim),
        in_specs=[full(a) for a in args],
        out_specs=[pl.BlockSpec((n, 1), lambda: (0, 0)),
                   pl.BlockSpec((n, hdim), lambda: (0, 0)),
                   pl.BlockSpec((1, 1), lambda: (0, 0))],
        out_shape=[jax.ShapeDtypeStruct((n, 1), jnp.float32),
                   jax.ShapeDtypeStruct((n, hdim), jnp.float32),
                   jax.ShapeDtypeStruct((1, 1), jnp.float32)],
    )(*args)

    return (y, h, t2.reshape(1))


# speedup vs baseline: 44.7602x; 1.1247x over previous
import functools

import jax
import jax.numpy as jnp
from jax import lax
from jax.experimental import pallas as pl
from jax.experimental.pallas import tpu as pltpu
from jax.experimental.pallas import tpu_sc as plsc

NW = 32
NC = 2


def _sc_body(per_w, nacc, x_hbm, edge_hbm,
             s_out, deg_out, x_v, src_v, dst_v, s_part, deg_part,
             sbuf, dbuf, os_v, od_v, sem, s_all, deg_all):
    c = lax.axis_index("c")
    s = lax.axis_index("s")
    wid = s * NC + c
    cp1 = pltpu.async_copy(x_hbm, x_v, sem)
    cp2 = pltpu.async_copy(edge_hbm.at[0, pl.ds(wid * per_w, per_w)], src_v, sem)
    cp3 = pltpu.async_copy(edge_hbm.at[1, pl.ds(wid * per_w, per_w)], dst_v, sem)

    zeros16 = jnp.zeros((16,), jnp.float32)

    @plsc.parallel_loop(0, nacc // 16, unroll=8)
    def _zero(j):
        sl = pl.ds(j * 16, 16)
        s_part[sl] = zeros16
        deg_part[sl] = zeros16

    cp1.wait()
    cp2.wait()
    cp3.wait()

    ones16 = jnp.ones((16,), jnp.float32)

    def one_vreg(j):
        base = j * 16
        idx_s = src_v[pl.ds(base, 16)]
        idx_d = dst_v[pl.ds(base, 16)]
        vals = plsc.load_gather(x_v, [idx_s])
        plsc.addupdate_scatter(s_part, [idx_d], vals)
        plsc.addupdate_scatter(deg_part, [idx_d], ones16)

    plsc.parallel_loop(0, per_w // 16, unroll=8)(one_vreg)

    pltpu.sync_copy(s_part, s_all.at[s])
    pltpu.sync_copy(deg_part, deg_all.at[s])
    plsc.subcore_barrier()

    w = nacc // 16
    pltpu.sync_copy(s_all.at[:, pl.ds(s * w, w)], sbuf)
    pltpu.sync_copy(deg_all.at[:, pl.ds(s * w, w)], dbuf)

    def red(v, carry):
        sl = pl.ds(v * 16, 16)
        accs = sbuf[0, sl]
        accd = dbuf[0, sl]
        for k in range(1, 16):
            accs = accs + sbuf[k, sl]
            accd = accd + dbuf[k, sl]
        os_v[sl] = accs
        od_v[sl] = accd
        return carry

    lax.fori_loop(0, w // 16, red, 0)
    pltpu.sync_copy(os_v, s_out.at[c, pl.ds(s * w, w)])
    pltpu.sync_copy(od_v, deg_out.at[c, pl.ds(s * w, w)])


def _segment_sums(x_flat, edge_index, n):
    e = edge_index.shape[1]
    per_w = e // NW
    nacc = -(-n // 256) * 256
    w = nacc // 16

    fn = pl.kernel(
        functools.partial(_sc_body, per_w, nacc),
        out_type=[jax.ShapeDtypeStruct((NC, nacc), jnp.float32),
                  jax.ShapeDtypeStruct((NC, nacc), jnp.float32)],
        mesh=plsc.VectorSubcoreMesh(core_axis_name="c", subcore_axis_name="s"),
        compiler_params=pltpu.CompilerParams(needs_layout_passes=False,
                                             use_tc_tiling_on_sc=False,
                                             skip_device_barrier=True),
        scratch_types=[
            pltpu.VMEM((x_flat.shape[0],), jnp.float32),
            pltpu.VMEM((per_w,), jnp.int32),
            pltpu.VMEM((per_w,), jnp.int32),
            pltpu.VMEM((nacc,), jnp.float32),
            pltpu.VMEM((nacc,), jnp.float32),
            pltpu.VMEM((16, w), jnp.float32),
            pltpu.VMEM((16, w), jnp.float32),
            pltpu.VMEM((w,), jnp.float32),
            pltpu.VMEM((w,), jnp.float32),
            pltpu.SemaphoreType.DMA,
            pltpu.VMEM_SHARED((16, nacc), jnp.float32),
            pltpu.VMEM_SHARED((16, nacc), jnp.float32),
        ],
    )
    return fn(x_flat, edge_index)


def _tc_body(n, hdim, x_r, sp_r, dp_r, u_r, be_r, ws_r, wn_r,
             bp_r, wd_r, bd_r, wt_r, bt_r, y_r, h_r, t_r):
    f32 = jnp.float32
    dnT = (((0,), (1,)), ((), ()))
    dn0 = (((0,), (0,)), ((), ()))
    u = u_r[...]
    be = be_r[...]
    ws = ws_r[...]
    wn = wn_r[...]
    a1 = lax.dot_general(ws, u, dnT, preferred_element_type=f32)
    a2 = lax.dot_general(wn, u, dnT, preferred_element_type=f32)
    a3 = lax.dot_general(wn, be, dnT, preferred_element_type=f32)
    a4 = (lax.dot_general(ws, be, dnT, preferred_element_type=f32)
          + lax.dot_general(bp_r[...], jnp.ones((1, 1), f32), dnT,
                            preferred_element_type=f32))

    x_row = x_r[...].reshape(1, n)
    s_row = sp_r[0:1, :n] + sp_r[1:2, :n]
    deg = dp_r[0:1, :n] + dp_r[1:2, :n]
    cde = jnp.maximum(deg, 1.0)
    r_row = s_row / cde
    m_row = jnp.minimum(deg, 1.0)

    pre = a1 * x_row + a2 * r_row + a3 * m_row + a4
    hT = jnp.maximum(pre, 0.0)

    eye = jnp.eye(hdim, dtype=f32)
    h = lax.dot_general(hT, eye, dn0, preferred_element_type=f32)
    h_r[...] = h

    wd = wd_r[...]
    wd1 = wd[:hdim, :]
    wd2 = wd[hdim:, :]
    c1 = lax.dot_general(u, wd2, (((1,), (0,)), ((), ())),
                         preferred_element_type=f32)
    c0 = lax.dot_general(be, wd2, (((1,), (0,)), ((), ())),
                         preferred_element_type=f32)
    x_col = lax.dot_general(x_row, jnp.ones((1, 1), f32), dn0,
                            preferred_element_type=f32)
    logits = (jnp.dot(h, wd1, preferred_element_type=f32)
              + x_col * c1 + c0 + bd_r[...])
    y_r[...] = jax.nn.sigmoid(logits)

    psum = jnp.sum(hT, axis=1, keepdims=True)
    pmax = jnp.max(hT, axis=1, keepdims=True)
    wt = wt_r[...]
    tt = (lax.dot_general(pmax, wt[:hdim, :], dn0, preferred_element_type=f32)
          + lax.dot_general(psum * (1.0 / n), wt[hdim:, :], dn0,
                            preferred_element_type=f32)
          + bt_r[...])
    t_r[...] = jax.nn.sigmoid(tt)


def kernel(x, edge_index, W_enc, b_enc, W_self, W_neigh, b_proc, W_dec,
           b_dec, W_term, b_term):
    n = x.shape[0]
    hdim = W_self.shape[0]
    x_flat = x.reshape(n)

    s_parts, deg_parts = _segment_sums(x_flat, edge_index, n)

    u = W_enc[0:1, :]
    be = b_enc.reshape(1, hdim)
    bp = b_proc.reshape(1, hdim)
    bd = b_dec.reshape(1, 1)
    bt = b_term.reshape(1, 1)

    full = lambda a: pl.BlockSpec(a.shape, lambda: (0,) * a.ndim)
    args = (x_flat, s_parts, deg_parts, u, be, W_self, W_neigh, bp,
            W_dec, bd, W_term, bt)
    y, h, t2 = pl.pallas_call(
        functools.partial(_tc_body, n, hdim),
        in_specs=[full(a) for a in args],
        out_specs=[pl.BlockSpec((n, 1), lambda: (0, 0)),
                   pl.BlockSpec((n, hdim), lambda: (0, 0)),
                   pl.BlockSpec((1, 1), lambda: (0, 0))],
        out_shape=[jax.ShapeDtypeStruct((n, 1), jnp.float32),
                   jax.ShapeDtypeStruct((n, hdim), jnp.float32),
                   jax.ShapeDtypeStruct((1, 1), jnp.float32)],
    )(*args)

    return (y, h, t2.reshape(1))
